# baseline (speedup 1.0000x reference)
import functools

import jax
import jax.numpy as jnp
from jax import lax
from jax.experimental import pallas as pl
from jax.experimental.pallas import tpu as pltpu
from jax.experimental.pallas import tpu_sc as plsc

_N = 256
_D = 4096
_K = 5
_INV_TEMPERATURE = 1.0 / 0.07
_CHUNK = 2048
_STEPS = _D // _CHUNK

_NC, _NS, _L = 2, 16, 16
_NW = _NC * _NS
_ROWS_PER = _N // _NW


def _gram(x):
    return jax.lax.dot_general(
        x, x, (((1,), (1,)), ((), ())), preferred_element_type=jnp.float32
    )


def _tc_kernel(features_ref, feat_t_g_ref, sim_ref, logp_ref, sim_acc, adc_acc):
    i = pl.program_id(0)

    g = feat_t_g_ref[...]
    f = features_ref[...]

    @pl.when(i == 0)
    def _init():
        sim_acc[...] = _gram(g)
        adc_acc[...] = _gram(f)

    @pl.when(i > 0)
    def _accum():
        sim_acc[...] += _gram(g)
        adc_acc[...] += _gram(f)

    @pl.when(i == _STEPS - 1)
    def _finish():
        col = jax.lax.broadcasted_iota(jnp.int32, (_N, _N), 1)
        row = jax.lax.broadcasted_iota(jnp.int32, (_N, _N), 0)
        off_diag = (row != col).astype(jnp.float32)

        adc = adc_acc[...] * _INV_TEMPERATURE
        logits_max = jnp.max(adc, axis=1, keepdims=True)
        logits = adc - logits_max

        exp_sum = jnp.sum(jnp.exp(logits) * off_diag, axis=1, keepdims=True)
        logp_ref[...] = logits - jnp.log(exp_sum)
        sim_ref[...] = sim_acc[...]


_sc_mesh = plsc.VectorSubcoreMesh(core_axis_name="c", subcore_axis_name="s")


@functools.partial(
    pl.kernel,
    out_type=jax.ShapeDtypeStruct((_N, _L), jnp.float32),
    mesh=_sc_mesh,
    scratch_types=[
        pltpu.VMEM((_ROWS_PER, _N), jnp.float32),
        pltpu.VMEM((_ROWS_PER, _N), jnp.float32),
        pltpu.VMEM((_ROWS_PER, _L), jnp.float32),
    ],
    compiler_params=pltpu.CompilerParams(needs_layout_passes=False),
)
def _sc_topk_loss(sim_hbm, logp_hbm, out_hbm, simbuf, logpbuf, outbuf):
    wid = lax.axis_index("s") * _NC + lax.axis_index("c")
    base = wid * _ROWS_PER
    pltpu.sync_copy(sim_hbm.at[pl.ds(base, _ROWS_PER), :], simbuf)
    pltpu.sync_copy(logp_hbm.at[pl.ds(base, _ROWS_PER), :], logpbuf)

    lane = lax.iota(jnp.int32, _L)
    lane0 = lane == 0
    neg_inf = jnp.full((_L,), -jnp.inf, jnp.float32)

    def _splat_last(v):
        head = jnp.where(lane0, lax.rev(v, (0,)), neg_inf)
        return plsc.cummax(head)

    def _maxsplat(v):
        return _splat_last(plsc.cummax(v))

    def _sumsplat(v):
        return _splat_last(plsc.cumsum(v))

    def _iminsplat(v):
        return (-_splat_last(plsc.cummax((-v).astype(jnp.float32)))).astype(
            jnp.int32
        )

    for r in range(_ROWS_PER):
        rowid = base + r
        selidx = jnp.zeros((_L,), jnp.int32)
        for it in range(_K):
            m = neg_inf
            for c in range(_N // _L):
                m = jnp.maximum(m, simbuf[r, pl.ds(c * _L, _L)])
            ms = _maxsplat(m)
            idxacc = jnp.full((_L,), _N, jnp.int32)
            for c in range(_N // _L):
                chunk = simbuf[r, pl.ds(c * _L, _L)]
                idxacc = jnp.minimum(
                    idxacc, jnp.where(chunk == ms, lane + c * _L, _N)
                )
            first = _iminsplat(idxacc)
            plsc.store_scatter(
                simbuf,
                [jnp.full((_L,), r, jnp.int32), first],
                neg_inf,
                mask=lane0,
            )
            selidx = jnp.where(lane == it, first, selidx)

        vals = plsc.load_gather(
            logpbuf,
            [jnp.full((_L,), r, jnp.int32), selidx],
        )
        valid = (lane < _K) & (selidx != rowid)
        s = _sumsplat(jnp.where(valid, vals, 0.0))
        cnt = _sumsplat(jnp.where(valid, 1.0, 0.0))
        denom = jnp.where(cnt == 0.0, 1.0, cnt)
        outbuf[r, :] = -(s / denom)

    pltpu.sync_copy(outbuf, out_hbm.at[pl.ds(base, _ROWS_PER), :])


@jax.jit
def kernel(features, feat_t_g):
    sim, logp = pl.pallas_call(
        _tc_kernel,
        grid=(_STEPS,),
        in_specs=[
            pl.BlockSpec((_N, _CHUNK), lambda i: (0, i)),
            pl.BlockSpec((_N, _CHUNK), lambda i: (0, i)),
        ],
        out_specs=[
            pl.BlockSpec((_N, _N), lambda i: (0, 0)),
            pl.BlockSpec((_N, _N), lambda i: (0, 0)),
        ],
        out_shape=[
            jax.ShapeDtypeStruct((_N, _N), jnp.float32),
            jax.ShapeDtypeStruct((_N, _N), jnp.float32),
        ],
        scratch_shapes=[
            pltpu.VMEM((_N, _N), jnp.float32),
            pltpu.VMEM((_N, _N), jnp.float32),
        ],
        compiler_params=pltpu.CompilerParams(
            dimension_semantics=("arbitrary",),
        ),
    )(features, feat_t_g)
    row_losses = _sc_topk_loss(sim, logp)
    return jnp.mean(row_losses[:, 0])


# speedup vs baseline: 4.7342x; 4.7342x over previous
import jax
import jax.numpy as jnp
from jax.experimental import pallas as pl
from jax.experimental.pallas import tpu as pltpu

_N = 256
_D = 4096
_K = 5
_INV_TEMPERATURE = 1.0 / 0.07
_CHUNK = 2048
_STEPS = _D // _CHUNK


def _gram(x):
    return jax.lax.dot_general(
        x, x, (((1,), (1,)), ((), ())), preferred_element_type=jnp.float32
    )


def _nncon_loss_kernel(features_ref, feat_t_g_ref, out_ref, sim_acc, adc_acc):
    i = pl.program_id(0)

    g = feat_t_g_ref[...]
    f = features_ref[...]

    @pl.when(i == 0)
    def _init():
        sim_acc[...] = _gram(g)
        adc_acc[...] = _gram(f)

    @pl.when(i > 0)
    def _accum():
        sim_acc[...] += _gram(g)
        adc_acc[...] += _gram(f)

    @pl.when(i == _STEPS - 1)
    def _finish():
        sim = sim_acc[...]
        col = jax.lax.broadcasted_iota(jnp.int32, (_N, _N), 1)

        work = sim
        mask = jnp.zeros((_N, _N), dtype=jnp.float32)
        for _ in range(_K):
            row_max = jnp.max(work, axis=1, keepdims=True)
            at_max = work == row_max
            first = jnp.min(jnp.where(at_max, col, _N), axis=1, keepdims=True)
            sel = col == first
            mask = mask + sel.astype(jnp.float32)
            work = jnp.where(sel, -jnp.inf, work)

        row = jax.lax.broadcasted_iota(jnp.int32, (_N, _N), 0)
        off_diag = (row != col).astype(jnp.float32)
        mask = mask * off_diag

        adc = adc_acc[...] * _INV_TEMPERATURE
        logits_max = jnp.max(adc, axis=1, keepdims=True)
        logits = adc - logits_max

        exp_sum = jnp.sum(jnp.exp(logits) * off_diag, axis=1, keepdims=True)
        log_es = jnp.log(exp_sum)[:, 0]

        msum = jnp.sum(mask, axis=1)
        denom = jnp.where(msum == 0.0, 1.0, msum)
        mean_log_prob_pos = (jnp.sum(mask * logits, axis=1) - log_es * msum) / denom

        out_ref[...] = (-jnp.sum(mean_log_prob_pos) / _N).reshape(1, 1)


@jax.jit
def kernel(features, feat_t_g):
    out = pl.pallas_call(
        _nncon_loss_kernel,
        grid=(_STEPS,),
        in_specs=[
            pl.BlockSpec((_N, _CHUNK), lambda i: (0, i)),
            pl.BlockSpec((_N, _CHUNK), lambda i: (0, i)),
        ],
        out_specs=pl.BlockSpec((1, 1), lambda i: (0, 0)),
        out_shape=jax.ShapeDtypeStruct((1, 1), jnp.float32),
        scratch_shapes=[
            pltpu.VMEM((_N, _N), jnp.float32),
            pltpu.VMEM((_N, _N), jnp.float32),
        ],
        compiler_params=pltpu.CompilerParams(
            dimension_semantics=("arbitrary",),
        ),
    )(features, feat_t_g)
    return out[0, 0]
